# baseline (speedup 1.0000x reference)
import jax
import jax.numpy as jnp
from jax.experimental import pallas as pl
from jax.experimental.pallas import tpu as pltpu

_N = 4096
_D = 2048
_TOPK = 20
_BR = 256
_BC = 256
_NEG = -3.0
_EPS = 1e-8


def _norm_kernel(x_ref, col_ref, row_ref):
    x = x_ref[...]
    inv = 1.0 / jnp.maximum(jnp.sqrt(jnp.sum(x * x, axis=1, keepdims=True)), _EPS)
    col_ref[...] = inv
    row_ref[...] = inv.reshape(1, -1)


def _topk_iter(work_ref, edge_ref, col):
    work = work_ref[...]
    m = jnp.max(work, axis=1, keepdims=True)
    cand = jnp.where(work == m, col, _N)
    amin = jnp.min(cand, axis=1, keepdims=True)
    sel = col == amin
    edge_ref[...] = jnp.where(sel, m, edge_ref[...])
    work_ref[...] = jnp.where(sel, _NEG, work)


def _simtopk_kernel(a_ref, b_ref, inva_ref, invb_ref, adj_ref, edge_ref,
                    acc0, acc1):
    i = pl.program_id(0)
    j = pl.program_id(1)
    ni = _N // _BR
    nj = _N // _BC
    base = _TOPK // nj
    extra = _TOPK % nj

    @pl.when(i < ni)
    def _():
        sim = jax.lax.dot_general(
            a_ref[...], b_ref[...], (((1,), (1,)), ((), ())),
            preferred_element_type=jnp.float32)
        sim = sim * inva_ref[...] * invb_ref[...]
        rows = jax.lax.broadcasted_iota(jnp.int32, sim.shape, 0)
        cols = jax.lax.broadcasted_iota(jnp.int32, sim.shape, 1)
        sim = jnp.where((i == j) & (rows == cols), 0.0, sim)

        @pl.when(i % 2 == 0)
        def _():
            acc0[:, pl.ds(j * _BC, _BC)] = sim

        @pl.when(i % 2 == 1)
        def _():
            acc1[:, pl.ds(j * _BC, _BC)] = sim

    @pl.when(i > 0)
    def _():
        col = jax.lax.broadcasted_iota(jnp.int32, (_BR, _N), 1)

        @pl.when(j == 0)
        def _():
            edge_ref[...] = jnp.zeros((_BR, _N), jnp.float32)

        def run(work_ref):
            for _ in range(base):
                _topk_iter(work_ref, edge_ref, col)

            if extra:
                @pl.when(j < extra)
                def _():
                    _topk_iter(work_ref, edge_ref, col)

        @pl.when(i % 2 == 0)
        def _():
            run(acc1)

        @pl.when(i % 2 == 1)
        def _():
            run(acc0)

        @pl.when(j == nj - 1)
        def _():
            adj_ref[...] = jnp.maximum(edge_ref[...], 0.0)


def kernel(h_style):
    inv_col, inv_row = pl.pallas_call(
        _norm_kernel,
        grid=(_N // _BR,),
        in_specs=[pl.BlockSpec((_BR, _D), lambda i: (i, 0))],
        out_specs=[
            pl.BlockSpec((_BR, 1), lambda i: (i, 0)),
            pl.BlockSpec((1, _BR), lambda i: (0, i)),
        ],
        out_shape=[
            jax.ShapeDtypeStruct((_N, 1), jnp.float32),
            jax.ShapeDtypeStruct((1, _N), jnp.float32),
        ],
    )(h_style)

    ni = _N // _BR
    adj, edge = pl.pallas_call(
        _simtopk_kernel,
        grid=(ni + 1, _N // _BC),
        in_specs=[
            pl.BlockSpec((_BR, _D), lambda i, j: (jnp.minimum(i, ni - 1), 0)),
            pl.BlockSpec((_BC, _D), lambda i, j: (j, 0)),
            pl.BlockSpec((_BR, 1), lambda i, j: (jnp.minimum(i, ni - 1), 0)),
            pl.BlockSpec((1, _BC), lambda i, j: (0, j)),
        ],
        out_specs=[
            pl.BlockSpec((_BR, _N), lambda i, j: (jnp.maximum(i, 1) - 1, 0)),
            pl.BlockSpec((_BR, _N), lambda i, j: (jnp.maximum(i, 1) - 1, 0)),
        ],
        out_shape=[
            jax.ShapeDtypeStruct((_N, _N), jnp.float32),
            jax.ShapeDtypeStruct((_N, _N), jnp.float32),
        ],
        scratch_shapes=[
            pltpu.VMEM((_BR, _N), jnp.float32),
            pltpu.VMEM((_BR, _N), jnp.float32),
        ],
        compiler_params=pltpu.CompilerParams(
            dimension_semantics=("arbitrary", "arbitrary")),
    )(h_style, h_style, inv_col, inv_row)
    return adj, edge[..., None]


# speedup vs baseline: 1.4489x; 1.4489x over previous
import jax
import jax.numpy as jnp
from jax.experimental import pallas as pl
from jax.experimental.pallas import tpu as pltpu

_N = 4096
_D = 2048
_TOPK = 20
_BR = 256
_BC = 256
_NEG = -3.0
_EPS = 1e-8


def _norm_kernel(x_ref, o_ref):
    x = x_ref[...]
    n = jnp.sqrt(jnp.sum(x * x, axis=1, keepdims=True))
    o_ref[...] = x / jnp.maximum(n, _EPS)


def _topk_iter(work_ref):
    work = work_ref[...]
    m = jnp.max(work, axis=1, keepdims=True)
    work_ref[...] = jnp.where(work == m, m - 4.0, work)


def _simtopk_kernel(a_ref, b_ref, adj_ref, edge_ref, acc0, acc1):
    i = pl.program_id(0)
    j = pl.program_id(1)
    ni = _N // _BR
    nj = _N // _BC
    base = _TOPK // nj
    extra = _TOPK % nj

    @pl.when(i < ni)
    def _():
        sim = jax.lax.dot_general(
            a_ref[...], b_ref[...], (((1,), (1,)), ((), ())),
            preferred_element_type=jnp.float32)
        rows = jax.lax.broadcasted_iota(jnp.int32, sim.shape, 0)
        cols = jax.lax.broadcasted_iota(jnp.int32, sim.shape, 1)
        sim = jnp.where((i == j) & (rows == cols), 0.0, sim)

        @pl.when(i % 2 == 0)
        def _():
            acc0[:, pl.ds(j * _BC, _BC)] = sim

        @pl.when(i % 2 == 1)
        def _():
            acc1[:, pl.ds(j * _BC, _BC)] = sim

    @pl.when(i > 0)
    def _():
        def run(work_ref):
            for _ in range(base):
                _topk_iter(work_ref)

            if extra:
                @pl.when(j < extra)
                def _():
                    _topk_iter(work_ref)

            @pl.when(j == nj - 1)
            def _():
                work = work_ref[...]
                edge = jnp.where(work < -2.0, work + 4.0, 0.0)
                edge_ref[...] = edge
                adj_ref[...] = jnp.maximum(edge, 0.0)

        @pl.when(i % 2 == 0)
        def _():
            run(acc1)

        @pl.when(i % 2 == 1)
        def _():
            run(acc0)


def kernel(h_style):
    hn = pl.pallas_call(
        _norm_kernel,
        grid=(_N // _BR,),
        in_specs=[pl.BlockSpec((_BR, _D), lambda i: (i, 0))],
        out_specs=pl.BlockSpec((_BR, _D), lambda i: (i, 0)),
        out_shape=jax.ShapeDtypeStruct((_N, _D), jnp.float32),
    )(h_style)

    ni = _N // _BR
    adj, edge = pl.pallas_call(
        _simtopk_kernel,
        grid=(ni + 1, _N // _BC),
        in_specs=[
            pl.BlockSpec((_BR, _D), lambda i, j: (jnp.minimum(i, ni - 1), 0)),
            pl.BlockSpec((_BC, _D), lambda i, j: (j, 0)),
        ],
        out_specs=[
            pl.BlockSpec((_BR, _N), lambda i, j: (jnp.maximum(i, 1) - 1, 0)),
            pl.BlockSpec((_BR, _N), lambda i, j: (jnp.maximum(i, 1) - 1, 0)),
        ],
        out_shape=[
            jax.ShapeDtypeStruct((_N, _N), jnp.float32),
            jax.ShapeDtypeStruct((_N, _N), jnp.float32),
        ],
        scratch_shapes=[
            pltpu.VMEM((_BR, _N), jnp.float32),
            pltpu.VMEM((_BR, _N), jnp.float32),
        ],
        compiler_params=pltpu.CompilerParams(
            dimension_semantics=("arbitrary", "arbitrary")),
    )(hn, hn)
    return adj, edge[..., None]
